# baseline (speedup 1.0000x reference)
import jax
import jax.numpy as jnp
from jax import lax
from jax.experimental import pallas as pl
from jax.experimental.pallas import tpu as pltpu
from jax.experimental.pallas import tpu_sc as plsc

D = 16
NC, NS = 2, 16
NW = NC * NS
N = 100000
BPW = 3200
LAST = N - (NW - 1) * BPW
CHUNKS_FULL = (1024, 1024, 1024, 128)
CHUNKS_LAST = (800,)
CMAX = 1024


def _run(base, chunks, idx_in_hbm, idx_out_hbm, table_hbm, out_hbm,
         idx_in_v, idx_out_v, rows_in_v, rows_out_v, out_v, sem):
    cnt = sum(chunks)
    pltpu.sync_copy(idx_in_hbm.at[pl.ds(base, cnt)], idx_in_v.at[pl.ds(0, cnt)])
    pltpu.sync_copy(idx_out_hbm.at[pl.ds(base, cnt)], idx_out_v.at[pl.ds(0, cnt)])

    off = 0
    for ch in chunks:
        cp_a = pltpu.async_copy(table_hbm.at[idx_in_v.at[pl.ds(off, ch)]],
                                rows_in_v.at[pl.ds(0, ch)], sem)
        cp_b = pltpu.async_copy(table_hbm.at[idx_out_v.at[pl.ds(off, ch)]],
                                rows_out_v.at[pl.ds(0, ch)], sem)
        cp_a.wait()
        cp_b.wait()

        @pl.loop(0, ch, unroll=8)
        def _add(j):
            s = rows_in_v[j, :] + rows_out_v[j, :]
            out_v[j >> 3, pl.ds((j & 7) * D, D)] = s

        pltpu.sync_copy(out_v.at[pl.ds(0, ch // 8)],
                        out_hbm.at[pl.ds((base + off) // 8, ch // 8)])
        off += ch


def _body(idx_in_hbm, idx_out_hbm, table_hbm, out_hbm,
          idx_in_v, idx_out_v, rows_in_v, rows_out_v, out_v, sem):
    c = lax.axis_index("c")
    s = lax.axis_index("s")
    wid = s * NC + c
    args = (idx_in_hbm, idx_out_hbm, table_hbm, out_hbm,
            idx_in_v, idx_out_v, rows_in_v, rows_out_v, out_v, sem)

    @pl.when(wid < NW - 1)
    def _full():
        _run(wid * BPW, CHUNKS_FULL, *args)

    @pl.when(wid == NW - 1)
    def _tail():
        _run((NW - 1) * BPW, CHUNKS_LAST, *args)


@jax.jit
def _degree_encode(idx_in, idx_out, table):
    mesh = plsc.VectorSubcoreMesh(core_axis_name="c", subcore_axis_name="s")
    f = pl.kernel(
        _body,
        out_type=jax.ShapeDtypeStruct((N * D // 128, 128), jnp.float32),
        mesh=mesh,
        scratch_types=[
            pltpu.VMEM((BPW,), jnp.int32),
            pltpu.VMEM((BPW,), jnp.int32),
            pltpu.VMEM((CMAX, D), jnp.float32),
            pltpu.VMEM((CMAX, D), jnp.float32),
            pltpu.VMEM((CMAX // 8, 128), jnp.float32),
            pltpu.SemaphoreType.DMA,
        ],
        compiler_params=pltpu.CompilerParams(use_tc_tiling_on_sc=False),
    )
    return f(idx_in, idx_out, table)


def kernel(in_degree, out_degree, table):
    ii = in_degree.astype(jnp.int32)
    oo = out_degree.astype(jnp.int32)
    return _degree_encode(ii, oo, table).reshape(N, D)
